# baseline (speedup 1.0000x reference)
import functools

import jax
import jax.numpy as jnp
from jax import lax
from jax.experimental import pallas as pl
from jax.experimental.pallas import tpu as pltpu
from jax.experimental.pallas import tpu_sc as plsc

N = 10000
E = 160000
S2 = 2000
S1 = 400
G = 64

NPAD = 10240
EPAD = 163840
S2PAD = 2048
S1PAD = 512

NC = 2
NS = 16
NW = NC * NS

_MESH = plsc.VectorSubcoreMesh(core_axis_name="c", subcore_axis_name="s")
_SC_PARAMS = pltpu.CompilerParams(use_tc_tiling_on_sc=False)
F32 = jnp.float32


def _wid():
    return lax.axis_index("s") * NC + lax.axis_index("c")


@functools.partial(
    pl.kernel,
    out_type=jax.ShapeDtypeStruct((NPAD, 16), F32),
    mesh=_MESH,
    scratch_types=[
        pltpu.VMEM((320,), jnp.int32),
        pltpu.VMEM((320,), jnp.int32),
        pltpu.VMEM((2, 64, 16), F32),
        pltpu.SemaphoreType.DMA,
        pltpu.SemaphoreType.DMA,
        pltpu.SemaphoreType.DMA,
    ],
    compiler_params=_SC_PARAMS,
)
def _sc_embed(ztab, xtab, zidx, xidx, h0, zi_v, xi_v, r_v, zsem, asem, ssem):
    base = _wid() * 320
    pltpu.sync_copy(zidx.at[pl.ds(base, 320)], zi_v)
    pltpu.sync_copy(xidx.at[pl.ds(base, 320)], xi_v)

    def fire_zg(k):
        pltpu.async_copy(ztab.at[zi_v.at[pl.ds(k * 64, 64)]],
                         r_v.at[lax.rem(k, 2)], zsem)

    fire_zg(0)

    def step(k, _):
        b = lax.rem(k, 2)
        pltpu.make_async_copy(ztab.at[zi_v.at[pl.ds(0, 64)]],
                              r_v.at[b], zsem).wait()
        pltpu.async_copy(xtab.at[xi_v.at[pl.ds(k * 64, 64)]],
                         r_v.at[b], asem, add=True)

        @pl.when(k >= 1)
        def _():
            pltpu.make_async_copy(r_v.at[b],
                                  h0.at[pl.ds(base, 64)], ssem).wait()

        @pl.when(k < 4)
        def _():
            fire_zg(k + 1)

        pltpu.make_async_copy(xtab.at[xi_v.at[pl.ds(0, 64)]],
                              r_v.at[b], asem).wait()
        pltpu.async_copy(r_v.at[b], h0.at[pl.ds(base + k * 64, 64)], ssem)
        return 0

    lax.fori_loop(0, 5, step, 0)
    pltpu.make_async_copy(r_v.at[0], h0.at[pl.ds(base, 64)], ssem).wait()


def _make_sc_gather(C):
    @functools.partial(
        pl.kernel,
        out_type=jax.ShapeDtypeStruct((EPAD, 128), F32),
        mesh=_MESH,
        scratch_types=[
            pltpu.VMEM((5120,), jnp.int32),
            pltpu.VMEM((4, 128, C), F32),
            pltpu.SemaphoreType.DMA,
            pltpu.SemaphoreType.DMA,
        ],
        compiler_params=_SC_PARAMS,
    )
    def gather(h, src, out, i_v, r_v, gsem, ssem):
        base = _wid() * 5120
        pltpu.sync_copy(src.at[pl.ds(base, 5120)], i_v)

        def fire_g(k):
            pltpu.async_copy(h.at[i_v.at[pl.ds(k * 128, 128)]],
                             r_v.at[lax.rem(k, 4)], gsem)

        for k in range(2):
            fire_g(k)

        def step(k, _):
            b = lax.rem(k, 4)
            pltpu.make_async_copy(
                h.at[i_v.at[pl.ds(0, 128)]],
                r_v.at[b], gsem).wait()
            pltpu.async_copy(r_v.at[b],
                             out.at[pl.ds(base + k * 128, 128), pl.ds(0, C)],
                             ssem)

            @pl.when(k >= 2)
            def _():
                pltpu.make_async_copy(
                    r_v.at[0],
                    out.at[pl.ds(base, 128), pl.ds(0, C)], ssem).wait()

            @pl.when(k + 2 < 40)
            def _():
                fire_g(k + 2)

            return 0

        lax.fori_loop(0, 40, step, 0)
        for _k in range(2):
            pltpu.make_async_copy(
                r_v.at[0],
                out.at[pl.ds(base, 128), pl.ds(0, C)], ssem).wait()

    return gather


_sc_gather16 = _make_sc_gather(16)
_sc_gather32 = _make_sc_gather(32)


def _make_sc_scatter(n_in):
    rows_per_worker = NPAD // NS
    nchunk = EPAD // 128 // NS

    @functools.partial(
        pl.kernel,
        out_type=jax.ShapeDtypeStruct((NPAD, 128), F32),
        mesh=_MESH,
        scratch_types=[
            pltpu.VMEM_SHARED((NPAD, 64), F32),
            pltpu.VMEM((128, 64), F32),
            pltpu.VMEM((4, 128, 64), F32),
            pltpu.VMEM((nchunk, 128), jnp.int32),
            pltpu.SemaphoreType.DMA,
            pltpu.SemaphoreType.DMA,
        ],
        compiler_params=_SC_PARAMS,
    )
    def scatter(*args):
        msgs = args[:n_in]
        dst2 = args[n_in]
        out = args[n_in + 1]
        acc, zb, m_v, i_v, lsem, asem = args[n_in + 2:]
        cid = lax.axis_index("c")
        sid = lax.axis_index("s")

        def zrow(i, _):
            for q in range(4):
                zb[i, pl.ds(q * 16, 16)] = jnp.zeros((16,), F32)
            return 0

        lax.fori_loop(0, 128, zrow, 0)

        r0 = sid * rows_per_worker
        for j in range(rows_per_worker // 128):
            pltpu.sync_copy(zb, acc.at[pl.ds(r0 + j * 128, 128)])

        cbase = sid * nchunk
        pltpu.sync_copy(dst2.at[pl.ds(cbase, nchunk)], i_v)
        plsc.subcore_barrier()

        for msg in msgs:
            def fire_l(k):
                pltpu.async_copy(
                    msg.at[pl.ds((cbase + k) * 128, 128),
                           pl.ds(cid * 64, 64)],
                    m_v.at[lax.rem(k, 4)], lsem)

            for k in range(2):
                fire_l(k)

            def step(k, _):
                b = lax.rem(k, 4)
                pltpu.make_async_copy(
                    msg.at[pl.ds(cbase * 128, 128), pl.ds(0, 64)],
                    m_v.at[b], lsem).wait()
                pltpu.async_copy(m_v.at[b], acc.at[i_v.at[k]],
                                 asem, add=True)

                @pl.when(k >= 2)
                def _():
                    pltpu.make_async_copy(m_v.at[0],
                                          acc.at[i_v.at[0]], asem).wait()

                @pl.when(k + 2 < nchunk)
                def _():
                    fire_l(k + 2)

                return 0

            lax.fori_loop(0, nchunk, step, 0)
            for _k in range(2):
                pltpu.make_async_copy(m_v.at[0],
                                      acc.at[i_v.at[0]], asem).wait()

        plsc.subcore_barrier()
        pltpu.sync_copy(acc.at[pl.ds(r0, rows_per_worker)],
                        out.at[pl.ds(r0, rows_per_worker),
                               pl.ds(cid * 64, 64)])

    return scatter


_sc_scatter1 = _make_sc_scatter(1)
_sc_scatter2 = _make_sc_scatter(2)


@functools.partial(
    pl.kernel,
    out_type=jax.ShapeDtypeStruct((G, 128), F32),
    mesh=_MESH,
    scratch_types=[
        pltpu.VMEM_SHARED((S2PAD, 128), F32),
        pltpu.VMEM_SHARED((S2PAD, 16), F32),
        pltpu.VMEM_SHARED((S1PAD, 128), F32),
        pltpu.VMEM_SHARED((S1PAD, 16), F32),
        pltpu.VMEM_SHARED((128, 128), F32),
        pltpu.VMEM_SHARED((128, 16), F32),
        pltpu.VMEM((128, 128), F32),
        pltpu.VMEM((128, 16), F32),
        pltpu.VMEM((128, 16), F32),
        pltpu.VMEM((128,), jnp.int32),
        pltpu.VMEM((32, 128), F32),
        pltpu.VMEM((32, 16), F32),
        pltpu.VMEM((32, 16), F32),
        pltpu.VMEM((32,), jnp.int32),
        pltpu.VMEM((4, 128), F32),
        pltpu.VMEM((4, 16), F32),
    ],
    compiler_params=_SC_PARAMS,
)
def _sc_pool(h2, n2s2, s2s, s2g, out,
             sum1, cnt1, sum2, cnt2, sum3, cnt3,
             vb, cb, ob, ib, vb32, cb32, ob32, ib32, vb4, cb4):
    cid = lax.axis_index("c")
    sid = lax.axis_index("s")

    @pl.when(cid == 0)
    def _():
        def fill(i, _):
            for q in range(8):
                vb[i, pl.ds(q * 16, 16)] = jnp.zeros((16,), F32)
            cb[i] = jnp.zeros((16,), F32)
            ob[i] = jnp.ones((16,), F32)
            return 0

        lax.fori_loop(0, 128, fill, 0)

        def fill32(i, _):
            ob32[i] = jnp.ones((16,), F32)
            return 0

        lax.fori_loop(0, 32, fill32, 0)

        pltpu.sync_copy(vb, sum1.at[pl.ds(sid * 128, 128)])
        pltpu.sync_copy(cb, cnt1.at[pl.ds(sid * 128, 128)])
        pltpu.sync_copy(vb.at[pl.ds(0, 32)], sum2.at[pl.ds(sid * 32, 32)])
        pltpu.sync_copy(cb.at[pl.ds(0, 32)], cnt2.at[pl.ds(sid * 32, 32)])
        pltpu.sync_copy(vb.at[pl.ds(0, 8)], sum3.at[pl.ds(sid * 8, 8)])
        pltpu.sync_copy(cb.at[pl.ds(0, 8)], cnt3.at[pl.ds(sid * 8, 8)])
        plsc.subcore_barrier()

        def chunk(k, _):
            off = sid * 640 + k * 128
            pltpu.sync_copy(n2s2.at[pl.ds(off, 128)], ib)
            pltpu.sync_copy(h2.at[pl.ds(off, 128)], vb)
            pltpu.sync_copy(vb, sum1.at[ib], add=True)
            pltpu.sync_copy(ob, cnt1.at[ib], add=True)
            return 0

        lax.fori_loop(0, 5, chunk, 0)
        plsc.subcore_barrier()

        r0 = sid * 128
        pltpu.sync_copy(sum1.at[pl.ds(r0, 128)], vb)
        pltpu.sync_copy(cnt1.at[pl.ds(r0, 128)], cb)

        def mean1(r, _):
            inv = 1.0 / jnp.maximum(cb[r], 1.0)
            for q in range(8):
                vb[r, pl.ds(q * 16, 16)] = vb[r, pl.ds(q * 16, 16)] * inv
            return 0

        lax.fori_loop(0, 128, mean1, 0)
        pltpu.sync_copy(s2s.at[pl.ds(r0, 128)], ib)
        pltpu.sync_copy(vb, sum2.at[ib], add=True)
        pltpu.sync_copy(ob, cnt2.at[ib], add=True)
        plsc.subcore_barrier()

        r1 = sid * 32
        pltpu.sync_copy(sum2.at[pl.ds(r1, 32)], vb32)
        pltpu.sync_copy(cnt2.at[pl.ds(r1, 32)], cb32)

        def mean2(r, _):
            inv = 1.0 / jnp.maximum(cb32[r], 1.0)
            for q in range(8):
                vb32[r, pl.ds(q * 16, 16)] = vb32[r, pl.ds(q * 16, 16)] * inv
            return 0

        lax.fori_loop(0, 32, mean2, 0)
        pltpu.sync_copy(s2g.at[pl.ds(r1, 32)], ib32)
        pltpu.sync_copy(vb32, sum3.at[ib32], add=True)
        pltpu.sync_copy(ob32, cnt3.at[ib32], add=True)
        plsc.subcore_barrier()

        r2 = sid * 4
        pltpu.sync_copy(sum3.at[pl.ds(r2, 4)], vb4)
        pltpu.sync_copy(cnt3.at[pl.ds(r2, 4)], cb4)

        def mean3(r, _):
            inv = 1.0 / jnp.maximum(cb4[r], 1.0)
            for q in range(8):
                vb4[r, pl.ds(q * 16, 16)] = vb4[r, pl.ds(q * 16, 16)] * inv
            return 0

        lax.fori_loop(0, 4, mean3, 0)
        pltpu.sync_copy(vb4, out.at[pl.ds(r2, 4)])


def _make_tc_msg(cin, C, ngrp, n_out, T):
    grid = EPAD // T
    per_out = ngrp // n_out

    def body(*refs):
        hs_ref, ea_ref, WaT_ref, ba_ref, Wg_ref, Bbw_ref, Rm_ref = refs[:7]
        outs = refs[7:]
        pid = pl.program_id(0)
        hs = hs_ref[...][:, :cin]
        u = jnp.maximum(
            jnp.dot(ea_ref[...], WaT_ref[...],
                    preferred_element_type=F32) + ba_ref[...], 0.0)
        hm = jnp.dot(hs.astype(jnp.bfloat16), Rm_ref[...],
                     preferred_element_type=F32)
        u16 = u.astype(jnp.bfloat16)
        row = lax.broadcasted_iota(jnp.int32, (T, 1), 0) + pid * T
        for o in range(n_out):
            acc = None
            for gg in range(per_out):
                g = o * per_out + gg
                blk = jnp.dot(u16, Wg_ref[g], preferred_element_type=F32)
                term = blk * hm[:, g * 128:(g + 1) * 128]
                acc = term if acc is None else acc + term
            if o == 0:
                acc = acc + jnp.dot(hs, Bbw_ref[...],
                                    preferred_element_type=F32)
            outs[o][...] = jnp.where(row < E, acc, 0.0)

    return pl.pallas_call(
        body,
        grid=(grid,),
        in_specs=[
            pl.BlockSpec((T, 128), lambda i: (i, 0)),
            pl.BlockSpec((T, 5), lambda i: (i, 0)),
            pl.BlockSpec((5, 128), lambda i: (0, 0)),
            pl.BlockSpec((1, 128), lambda i: (0, 0)),
            pl.BlockSpec((ngrp, 128, 128), lambda i: (0, 0, 0)),
            pl.BlockSpec((cin, 128), lambda i: (0, 0)),
            pl.BlockSpec((cin, ngrp * 128), lambda i: (0, 0)),
        ],
        out_specs=[pl.BlockSpec((T, 128), lambda i: (i, 0))] * n_out,
        out_shape=[jax.ShapeDtypeStruct((EPAD, 128), F32)] * n_out,
    )


_tc_msg0 = _make_tc_msg(16, 32, 4, 1, 512)
_tc_msg1 = _make_tc_msg(32, 64, 16, 2, 512)


def _elu(v):
    return jnp.where(v > 0, v, jnp.exp(jnp.minimum(v, 0.0)) - 1.0)


def _make_tc_node(cin, C, T, out_wide):
    grid = NPAD // T
    nblk = 128 // C

    def body(h_ref, A_ref, rT_ref, b_ref, o_ref):
        pid = pl.program_id(0)
        v = jnp.dot(h_ref[...], rT_ref[...],
                    preferred_element_type=F32)
        A = A_ref[...]
        for j in range(nblk):
            v = v + A[:, j * C:(j + 1) * C]
        v = _elu(v + b_ref[...])
        row = lax.broadcasted_iota(jnp.int32, (T, 1), 0) + pid * T
        v = jnp.where(row < N, v, 0.0)
        if out_wide:
            o_ref[:, :C] = v
            o_ref[:, C:] = jnp.zeros((T, 128 - C), F32)
        else:
            o_ref[...] = v

    return pl.pallas_call(
        body,
        grid=(grid,),
        in_specs=[
            pl.BlockSpec((T, cin), lambda i: (i, 0)),
            pl.BlockSpec((T, 128), lambda i: (i, 0)),
            pl.BlockSpec((cin, C), lambda i: (0, 0)),
            pl.BlockSpec((1, C), lambda i: (0, 0)),
        ],
        out_specs=pl.BlockSpec((T, 128 if out_wide else C),
                               lambda i: (i, 0)),
        out_shape=jax.ShapeDtypeStruct((NPAD, 128 if out_wide else C), F32),
    )


_tc_node0 = _make_tc_node(16, 32, 1024, out_wide=False)
_tc_node1 = _make_tc_node(32, 64, 1024, out_wide=True)


def _tc_mlp(pooled, w1T, b1, w2T, b2, w3T, b3):
    def body(p_ref, w1_ref, b1_ref, w2_ref, b2_ref, w3_ref, b3_ref, o_ref):
        v = _elu(jnp.dot(p_ref[...][:, :64], w1_ref[...],
                         preferred_element_type=F32) + b1_ref[...])
        v = _elu(jnp.dot(v, w2_ref[...],
                         preferred_element_type=F32) + b2_ref[...])
        o_ref[...] = jnp.dot(v, w3_ref[...],
                             preferred_element_type=F32) + b3_ref[...]

    return pl.pallas_call(
        body,
        out_shape=jax.ShapeDtypeStruct((G, 1), F32),
    )(pooled, w1T, b1, w2T, b2, w3T, b3)


def kernel(x, z, edge_index, edge_attr, batch, node_to_subgraph2,
           subgraph2_to_subgraph, subgraph_to_graph,
           z_table, nt_table,
           W0a, b0a, W0b, b0b, root0, bias0,
           W1a, b1a, W1b, b1b, root1, bias1,
           fc1_w, fc1_b, fc2_w, fc2_b, fc3_w, fc3_b):
    i32 = jnp.int32

    ztab = jnp.pad(z_table, ((0, 0), (0, 8)))
    xtab = jnp.pad(nt_table, ((0, 3), (0, 8)))
    xtab = xtab.at[:5, 8].set(jnp.arange(5, dtype=F32))

    zidx = jnp.pad(z.astype(i32), (0, NPAD - N))
    xidx = jnp.pad(x.astype(i32), (0, NPAD - N))
    src = jnp.pad(edge_index[0].astype(i32), (0, EPAD - E))
    dst = jnp.pad(edge_index[1].astype(i32), (0, EPAD - E))
    dst2 = dst.reshape(EPAD // 128, 128)
    ea5 = jnp.pad(edge_attr, ((0, EPAD - E), (0, 0)))

    W0aT = W0a.T
    W1aT = W1a.T
    b0a_r = b0a.reshape(1, 128)
    b1a_r = b1a.reshape(1, 128)
    Wg0 = jnp.pad(W0b.reshape(9, 32, 128), ((0, 7), (0, 0), (0, 0)))
    Wg0 = Wg0.transpose(0, 2, 1).reshape(4, 4, 128, 32)
    Wg0 = Wg0.transpose(0, 2, 1, 3).reshape(4, 128, 128)
    Wg1 = W1b.reshape(32, 64, 128).transpose(0, 2, 1).reshape(16, 2, 128, 64)
    Wg1 = Wg1.transpose(0, 2, 1, 3).reshape(16, 128, 128)
    Wg0 = Wg0.astype(jnp.bfloat16)
    Wg1 = Wg1.astype(jnp.bfloat16)
    Bb0w = jnp.pad(jnp.pad(b0b.reshape(9, 32), ((0, 7), (0, 0))),
                   ((0, 0), (0, 96)))
    Bb1w = jnp.pad(b1b.reshape(32, 64), ((0, 0), (0, 64)))
    Rm0 = jnp.kron(jnp.eye(16, dtype=F32), jnp.ones((1, 32), F32))
    Rm1 = jnp.kron(jnp.eye(32, dtype=F32), jnp.ones((1, 64), F32))
    Rm0 = Rm0.astype(jnp.bfloat16)
    Rm1 = Rm1.astype(jnp.bfloat16)

    r0T = jnp.pad(root0.T, ((0, 7), (0, 0)))
    r1T = root1.T
    bias0_r = bias0.reshape(1, 32)
    bias1_r = bias1.reshape(1, 64)

    n2s2p = jnp.pad(node_to_subgraph2.astype(i32), (0, NPAD - N),
                    constant_values=S2)
    s2sp = jnp.pad(subgraph2_to_subgraph.astype(i32), (0, S2PAD - S2),
                   constant_values=S1)
    s2gp = jnp.pad(subgraph_to_graph.astype(i32), (0, S1PAD - S1),
                   constant_values=G)

    fc1T = fc1_w.T
    fc2T = fc2_w.T
    fc3T = fc3_w.T
    fb1 = fc1_b.reshape(1, 32)
    fb2 = fc2_b.reshape(1, 16)
    fb3 = fc3_b.reshape(1, 1)

    h0 = _sc_embed(ztab, xtab, zidx, xidx)
    h0s = _sc_gather16(h0, src)
    (m0,) = _tc_msg0(h0s, ea5, W0aT, b0a_r, Wg0, Bb0w, Rm0)
    A0 = _sc_scatter1(m0, dst2)
    h1 = _tc_node0(h0, A0, r0T, bias0_r)
    h1s = _sc_gather32(h1, src)
    m1a, m1b = _tc_msg1(h1s, ea5, W1aT, b1a_r, Wg1, Bb1w, Rm1)
    A1 = _sc_scatter2(m1a, m1b, dst2)
    h2 = _tc_node1(h1, A1, r1T, bias1_r)
    pooled = _sc_pool(h2, n2s2p, s2sp, s2gp)
    return _tc_mlp(pooled, fc1T, fb1, fc2T, fb2, fc3T, fb3)


# speedup vs baseline: 1.1389x; 1.1389x over previous
import functools

import jax
import jax.numpy as jnp
from jax import lax
from jax.experimental import pallas as pl
from jax.experimental.pallas import tpu as pltpu
from jax.experimental.pallas import tpu_sc as plsc

N = 10000
E = 160000
S2 = 2000
S1 = 400
G = 64

NPAD = 10240
EPAD = 163840
S2PAD = 2048
S1PAD = 512

NC = 2
NS = 16
NW = NC * NS

_MESH = plsc.VectorSubcoreMesh(core_axis_name="c", subcore_axis_name="s")
_SC_PARAMS = pltpu.CompilerParams(use_tc_tiling_on_sc=False)
F32 = jnp.float32


def _wid():
    return lax.axis_index("s") * NC + lax.axis_index("c")


@functools.partial(
    pl.kernel,
    out_type=jax.ShapeDtypeStruct((NPAD, 16), F32),
    mesh=_MESH,
    scratch_types=[
        pltpu.VMEM((320,), jnp.int32),
        pltpu.VMEM((320,), jnp.int32),
        pltpu.VMEM((2, 64, 16), F32),
        pltpu.SemaphoreType.DMA,
        pltpu.SemaphoreType.DMA,
        pltpu.SemaphoreType.DMA,
    ],
    compiler_params=_SC_PARAMS,
)
def _sc_embed(ztab, xtab, zidx, xidx, h0, zi_v, xi_v, r_v, zsem, asem, ssem):
    base = _wid() * 320
    pltpu.sync_copy(zidx.at[pl.ds(base, 320)], zi_v)
    pltpu.sync_copy(xidx.at[pl.ds(base, 320)], xi_v)

    def fire_zg(k):
        pltpu.async_copy(ztab.at[zi_v.at[pl.ds(k * 64, 64)]],
                         r_v.at[lax.rem(k, 2)], zsem)

    fire_zg(0)

    def step(k, _):
        b = lax.rem(k, 2)
        pltpu.make_async_copy(ztab.at[zi_v.at[pl.ds(0, 64)]],
                              r_v.at[b], zsem).wait()
        pltpu.async_copy(xtab.at[xi_v.at[pl.ds(k * 64, 64)]],
                         r_v.at[b], asem, add=True)

        @pl.when(k >= 1)
        def _():
            pltpu.make_async_copy(r_v.at[b],
                                  h0.at[pl.ds(base, 64)], ssem).wait()

        @pl.when(k < 4)
        def _():
            fire_zg(k + 1)

        pltpu.make_async_copy(xtab.at[xi_v.at[pl.ds(0, 64)]],
                              r_v.at[b], asem).wait()
        pltpu.async_copy(r_v.at[b], h0.at[pl.ds(base + k * 64, 64)], ssem)
        return 0

    lax.fori_loop(0, 5, step, 0)
    pltpu.make_async_copy(r_v.at[0], h0.at[pl.ds(base, 64)], ssem).wait()


def _make_sc_gather(C):
    @functools.partial(
        pl.kernel,
        out_type=jax.ShapeDtypeStruct((EPAD, 128), F32),
        mesh=_MESH,
        scratch_types=[
            pltpu.VMEM((5120,), jnp.int32),
            pltpu.VMEM((4, 128, C), F32),
            pltpu.SemaphoreType.DMA,
            pltpu.SemaphoreType.DMA,
        ],
        compiler_params=_SC_PARAMS,
    )
    def gather(h, src, out, i_v, r_v, gsem, ssem):
        base = _wid() * 5120
        pltpu.sync_copy(src.at[pl.ds(base, 5120)], i_v)

        def fire_g(k):
            pltpu.async_copy(h.at[i_v.at[pl.ds(k * 128, 128)]],
                             r_v.at[lax.rem(k, 4)], gsem)

        for k in range(2):
            fire_g(k)

        def step(k, _):
            b = lax.rem(k, 4)
            pltpu.make_async_copy(
                h.at[i_v.at[pl.ds(0, 128)]],
                r_v.at[b], gsem).wait()
            pltpu.async_copy(r_v.at[b],
                             out.at[pl.ds(base + k * 128, 128), pl.ds(0, C)],
                             ssem)

            @pl.when(k >= 2)
            def _():
                pltpu.make_async_copy(
                    r_v.at[0],
                    out.at[pl.ds(base, 128), pl.ds(0, C)], ssem).wait()

            @pl.when(k + 2 < 40)
            def _():
                fire_g(k + 2)

            return 0

        lax.fori_loop(0, 40, step, 0)
        for _k in range(2):
            pltpu.make_async_copy(
                r_v.at[0],
                out.at[pl.ds(base, 128), pl.ds(0, C)], ssem).wait()

    return gather


_sc_gather16 = _make_sc_gather(16)
_sc_gather32 = _make_sc_gather(32)


def _make_sc_scatter(n_in):
    rows_per_worker = NPAD // NS
    nchunk = EPAD // 128 // NS

    @functools.partial(
        pl.kernel,
        out_type=jax.ShapeDtypeStruct((NPAD, 128), F32),
        mesh=_MESH,
        scratch_types=[
            pltpu.VMEM_SHARED((NPAD, 64), F32),
            pltpu.VMEM((128, 64), F32),
            pltpu.VMEM((4, 128, 64), F32),
            pltpu.VMEM((nchunk, 128), jnp.int32),
            pltpu.SemaphoreType.DMA,
            pltpu.SemaphoreType.DMA,
        ],
        compiler_params=_SC_PARAMS,
    )
    def scatter(*args):
        msgs = args[:n_in]
        dst2 = args[n_in]
        out = args[n_in + 1]
        acc, zb, m_v, i_v, lsem, asem = args[n_in + 2:]
        cid = lax.axis_index("c")
        sid = lax.axis_index("s")

        def zrow(i, _):
            for q in range(4):
                zb[i, pl.ds(q * 16, 16)] = jnp.zeros((16,), F32)
            return 0

        lax.fori_loop(0, 128, zrow, 0)

        r0 = sid * rows_per_worker
        for j in range(rows_per_worker // 128):
            pltpu.sync_copy(zb, acc.at[pl.ds(r0 + j * 128, 128)])

        cbase = sid * nchunk
        pltpu.sync_copy(dst2.at[pl.ds(cbase, nchunk)], i_v)
        plsc.subcore_barrier()

        for msg in msgs:
            def fire_l(k):
                pltpu.async_copy(
                    msg.at[pl.ds((cbase + k) * 128, 128),
                           pl.ds(cid * 64, 64)],
                    m_v.at[lax.rem(k, 4)], lsem)

            for k in range(2):
                fire_l(k)

            def step(k, _):
                b = lax.rem(k, 4)
                pltpu.make_async_copy(
                    msg.at[pl.ds(cbase * 128, 128), pl.ds(0, 64)],
                    m_v.at[b], lsem).wait()
                pltpu.async_copy(m_v.at[b], acc.at[i_v.at[k]],
                                 asem, add=True)

                @pl.when(k >= 2)
                def _():
                    pltpu.make_async_copy(m_v.at[0],
                                          acc.at[i_v.at[0]], asem).wait()

                @pl.when(k + 2 < nchunk)
                def _():
                    fire_l(k + 2)

                return 0

            lax.fori_loop(0, nchunk, step, 0)
            for _k in range(2):
                pltpu.make_async_copy(m_v.at[0],
                                      acc.at[i_v.at[0]], asem).wait()

        plsc.subcore_barrier()
        pltpu.sync_copy(acc.at[pl.ds(r0, rows_per_worker)],
                        out.at[pl.ds(r0, rows_per_worker),
                               pl.ds(cid * 64, 64)])

    return scatter


_sc_scatter1 = _make_sc_scatter(1)
_sc_scatter2 = _make_sc_scatter(2)


@functools.partial(
    pl.kernel,
    out_type=jax.ShapeDtypeStruct((G, 128), F32),
    mesh=_MESH,
    scratch_types=[
        pltpu.VMEM_SHARED((S2PAD, 128), F32),
        pltpu.VMEM_SHARED((S2PAD, 16), F32),
        pltpu.VMEM_SHARED((S1PAD, 128), F32),
        pltpu.VMEM_SHARED((S1PAD, 16), F32),
        pltpu.VMEM_SHARED((128, 128), F32),
        pltpu.VMEM_SHARED((128, 16), F32),
        pltpu.VMEM((128, 128), F32),
        pltpu.VMEM((128, 16), F32),
        pltpu.VMEM((128, 16), F32),
        pltpu.VMEM((128,), jnp.int32),
        pltpu.VMEM((32, 128), F32),
        pltpu.VMEM((32, 16), F32),
        pltpu.VMEM((32, 16), F32),
        pltpu.VMEM((32,), jnp.int32),
        pltpu.VMEM((4, 128), F32),
        pltpu.VMEM((4, 16), F32),
    ],
    compiler_params=_SC_PARAMS,
)
def _sc_pool(h2, n2s2, s2s, s2g, out,
             sum1, cnt1, sum2, cnt2, sum3, cnt3,
             vb, cb, ob, ib, vb32, cb32, ob32, ib32, vb4, cb4):
    cid = lax.axis_index("c")
    sid = lax.axis_index("s")

    @pl.when(cid == 0)
    def _():
        def fill(i, _):
            for q in range(8):
                vb[i, pl.ds(q * 16, 16)] = jnp.zeros((16,), F32)
            cb[i] = jnp.zeros((16,), F32)
            ob[i] = jnp.ones((16,), F32)
            return 0

        lax.fori_loop(0, 128, fill, 0)

        def fill32(i, _):
            ob32[i] = jnp.ones((16,), F32)
            return 0

        lax.fori_loop(0, 32, fill32, 0)

        pltpu.sync_copy(vb, sum1.at[pl.ds(sid * 128, 128)])
        pltpu.sync_copy(cb, cnt1.at[pl.ds(sid * 128, 128)])
        pltpu.sync_copy(vb.at[pl.ds(0, 32)], sum2.at[pl.ds(sid * 32, 32)])
        pltpu.sync_copy(cb.at[pl.ds(0, 32)], cnt2.at[pl.ds(sid * 32, 32)])
        pltpu.sync_copy(vb.at[pl.ds(0, 8)], sum3.at[pl.ds(sid * 8, 8)])
        pltpu.sync_copy(cb.at[pl.ds(0, 8)], cnt3.at[pl.ds(sid * 8, 8)])
        plsc.subcore_barrier()

        def chunk(k, _):
            off = sid * 640 + k * 128
            pltpu.sync_copy(n2s2.at[pl.ds(off, 128)], ib)
            pltpu.sync_copy(h2.at[pl.ds(off, 128)], vb)
            pltpu.sync_copy(vb, sum1.at[ib], add=True)
            pltpu.sync_copy(ob, cnt1.at[ib], add=True)
            return 0

        lax.fori_loop(0, 5, chunk, 0)
        plsc.subcore_barrier()

        r0 = sid * 128
        pltpu.sync_copy(sum1.at[pl.ds(r0, 128)], vb)
        pltpu.sync_copy(cnt1.at[pl.ds(r0, 128)], cb)

        def mean1(r, _):
            inv = 1.0 / jnp.maximum(cb[r], 1.0)
            for q in range(8):
                vb[r, pl.ds(q * 16, 16)] = vb[r, pl.ds(q * 16, 16)] * inv
            return 0

        lax.fori_loop(0, 128, mean1, 0)
        pltpu.sync_copy(s2s.at[pl.ds(r0, 128)], ib)
        pltpu.sync_copy(vb, sum2.at[ib], add=True)
        pltpu.sync_copy(ob, cnt2.at[ib], add=True)
        plsc.subcore_barrier()

        r1 = sid * 32
        pltpu.sync_copy(sum2.at[pl.ds(r1, 32)], vb32)
        pltpu.sync_copy(cnt2.at[pl.ds(r1, 32)], cb32)

        def mean2(r, _):
            inv = 1.0 / jnp.maximum(cb32[r], 1.0)
            for q in range(8):
                vb32[r, pl.ds(q * 16, 16)] = vb32[r, pl.ds(q * 16, 16)] * inv
            return 0

        lax.fori_loop(0, 32, mean2, 0)
        pltpu.sync_copy(s2g.at[pl.ds(r1, 32)], ib32)
        pltpu.sync_copy(vb32, sum3.at[ib32], add=True)
        pltpu.sync_copy(ob32, cnt3.at[ib32], add=True)
        plsc.subcore_barrier()

        r2 = sid * 4
        pltpu.sync_copy(sum3.at[pl.ds(r2, 4)], vb4)
        pltpu.sync_copy(cnt3.at[pl.ds(r2, 4)], cb4)

        def mean3(r, _):
            inv = 1.0 / jnp.maximum(cb4[r], 1.0)
            for q in range(8):
                vb4[r, pl.ds(q * 16, 16)] = vb4[r, pl.ds(q * 16, 16)] * inv
            return 0

        lax.fori_loop(0, 4, mean3, 0)
        pltpu.sync_copy(vb4, out.at[pl.ds(r2, 4)])


def _make_tc_msg(cin, C, ngrp, n_out, T):
    grid = EPAD // T
    per_out = ngrp // n_out

    def body(*refs):
        hs_ref, ea_ref, WaT_ref, ba_ref, Wg_ref, Bbw_ref, Rm_ref = refs[:7]
        outs = refs[7:]
        pid = pl.program_id(0)
        hs = hs_ref[...][:, :cin]
        u = jnp.maximum(
            jnp.dot(ea_ref[...], WaT_ref[...],
                    preferred_element_type=F32) + ba_ref[...], 0.0)
        hm = jnp.dot(hs.astype(jnp.bfloat16), Rm_ref[...],
                     preferred_element_type=F32)
        u16 = u.astype(jnp.bfloat16)
        row = lax.broadcasted_iota(jnp.int32, (T, 1), 0) + pid * T
        for o in range(n_out):
            acc = None
            for gg in range(per_out):
                g = o * per_out + gg
                blk = jnp.dot(u16, Wg_ref[g], preferred_element_type=F32)
                term = blk * hm[:, g * 128:(g + 1) * 128]
                acc = term if acc is None else acc + term
            if o == 0:
                acc = acc + jnp.dot(hs, Bbw_ref[...],
                                    preferred_element_type=F32)
            outs[o][...] = jnp.where(row < E, acc, 0.0)

    return pl.pallas_call(
        body,
        grid=(grid,),
        in_specs=[
            pl.BlockSpec((T, 128), lambda i: (i, 0)),
            pl.BlockSpec((T, 5), lambda i: (i, 0)),
            pl.BlockSpec((5, 128), lambda i: (0, 0)),
            pl.BlockSpec((1, 128), lambda i: (0, 0)),
            pl.BlockSpec((ngrp, 128, 128), lambda i: (0, 0, 0)),
            pl.BlockSpec((cin, 128), lambda i: (0, 0)),
            pl.BlockSpec((cin, ngrp * 128), lambda i: (0, 0)),
        ],
        out_specs=[pl.BlockSpec((T, 128), lambda i: (i, 0))] * n_out,
        out_shape=[jax.ShapeDtypeStruct((EPAD, 128), F32)] * n_out,
    )


_tc_msg0 = _make_tc_msg(16, 32, 4, 1, 1024)
_tc_msg1 = _make_tc_msg(32, 64, 16, 2, 1024)


def _elu(v):
    return jnp.where(v > 0, v, jnp.exp(jnp.minimum(v, 0.0)) - 1.0)


def _make_tc_node(cin, C, T, out_wide):
    grid = NPAD // T
    nblk = 128 // C

    def body(h_ref, A_ref, rT_ref, b_ref, o_ref):
        pid = pl.program_id(0)
        v = jnp.dot(h_ref[...], rT_ref[...],
                    preferred_element_type=F32)
        A = A_ref[...]
        for j in range(nblk):
            v = v + A[:, j * C:(j + 1) * C]
        v = _elu(v + b_ref[...])
        row = lax.broadcasted_iota(jnp.int32, (T, 1), 0) + pid * T
        v = jnp.where(row < N, v, 0.0)
        if out_wide:
            o_ref[:, :C] = v
            o_ref[:, C:] = jnp.zeros((T, 128 - C), F32)
        else:
            o_ref[...] = v

    return pl.pallas_call(
        body,
        grid=(grid,),
        in_specs=[
            pl.BlockSpec((T, cin), lambda i: (i, 0)),
            pl.BlockSpec((T, 128), lambda i: (i, 0)),
            pl.BlockSpec((cin, C), lambda i: (0, 0)),
            pl.BlockSpec((1, C), lambda i: (0, 0)),
        ],
        out_specs=pl.BlockSpec((T, 128 if out_wide else C),
                               lambda i: (i, 0)),
        out_shape=jax.ShapeDtypeStruct((NPAD, 128 if out_wide else C), F32),
    )


_tc_node0 = _make_tc_node(16, 32, 1024, out_wide=False)
_tc_node1 = _make_tc_node(32, 64, 1024, out_wide=True)


def _tc_mlp(pooled, w1T, b1, w2T, b2, w3T, b3):
    def body(p_ref, w1_ref, b1_ref, w2_ref, b2_ref, w3_ref, b3_ref, o_ref):
        v = _elu(jnp.dot(p_ref[...][:, :64], w1_ref[...],
                         preferred_element_type=F32) + b1_ref[...])
        v = _elu(jnp.dot(v, w2_ref[...],
                         preferred_element_type=F32) + b2_ref[...])
        o_ref[...] = jnp.dot(v, w3_ref[...],
                             preferred_element_type=F32) + b3_ref[...]

    return pl.pallas_call(
        body,
        out_shape=jax.ShapeDtypeStruct((G, 1), F32),
    )(pooled, w1T, b1, w2T, b2, w3T, b3)


def kernel(x, z, edge_index, edge_attr, batch, node_to_subgraph2,
           subgraph2_to_subgraph, subgraph_to_graph,
           z_table, nt_table,
           W0a, b0a, W0b, b0b, root0, bias0,
           W1a, b1a, W1b, b1b, root1, bias1,
           fc1_w, fc1_b, fc2_w, fc2_b, fc3_w, fc3_b):
    i32 = jnp.int32

    ztab = jnp.pad(z_table, ((0, 0), (0, 8)))
    xtab = jnp.pad(nt_table, ((0, 3), (0, 8)))
    xtab = xtab.at[:5, 8].set(jnp.arange(5, dtype=F32))

    zidx = jnp.pad(z.astype(i32), (0, NPAD - N))
    xidx = jnp.pad(x.astype(i32), (0, NPAD - N))
    src = jnp.pad(edge_index[0].astype(i32), (0, EPAD - E))
    dst = jnp.pad(edge_index[1].astype(i32), (0, EPAD - E))
    dst2 = dst.reshape(EPAD // 128, 128)
    ea5 = jnp.pad(edge_attr, ((0, EPAD - E), (0, 0)))

    W0aT = W0a.T
    W1aT = W1a.T
    b0a_r = b0a.reshape(1, 128)
    b1a_r = b1a.reshape(1, 128)
    Wg0 = jnp.pad(W0b.reshape(9, 32, 128), ((0, 7), (0, 0), (0, 0)))
    Wg0 = Wg0.transpose(0, 2, 1).reshape(4, 4, 128, 32)
    Wg0 = Wg0.transpose(0, 2, 1, 3).reshape(4, 128, 128)
    Wg1 = W1b.reshape(32, 64, 128).transpose(0, 2, 1).reshape(16, 2, 128, 64)
    Wg1 = Wg1.transpose(0, 2, 1, 3).reshape(16, 128, 128)
    Wg0 = Wg0.astype(jnp.bfloat16)
    Wg1 = Wg1.astype(jnp.bfloat16)
    Bb0w = jnp.pad(jnp.pad(b0b.reshape(9, 32), ((0, 7), (0, 0))),
                   ((0, 0), (0, 96)))
    Bb1w = jnp.pad(b1b.reshape(32, 64), ((0, 0), (0, 64)))
    Rm0 = jnp.kron(jnp.eye(16, dtype=F32), jnp.ones((1, 32), F32))
    Rm1 = jnp.kron(jnp.eye(32, dtype=F32), jnp.ones((1, 64), F32))
    Rm0 = Rm0.astype(jnp.bfloat16)
    Rm1 = Rm1.astype(jnp.bfloat16)

    r0T = jnp.pad(root0.T, ((0, 7), (0, 0)))
    r1T = root1.T
    bias0_r = bias0.reshape(1, 32)
    bias1_r = bias1.reshape(1, 64)

    n2s2p = jnp.pad(node_to_subgraph2.astype(i32), (0, NPAD - N),
                    constant_values=S2)
    s2sp = jnp.pad(subgraph2_to_subgraph.astype(i32), (0, S2PAD - S2),
                   constant_values=S1)
    s2gp = jnp.pad(subgraph_to_graph.astype(i32), (0, S1PAD - S1),
                   constant_values=G)

    fc1T = fc1_w.T
    fc2T = fc2_w.T
    fc3T = fc3_w.T
    fb1 = fc1_b.reshape(1, 32)
    fb2 = fc2_b.reshape(1, 16)
    fb3 = fc3_b.reshape(1, 1)

    h0 = _sc_embed(ztab, xtab, zidx, xidx)
    h0s = _sc_gather16(h0, src)
    (m0,) = _tc_msg0(h0s, ea5, W0aT, b0a_r, Wg0, Bb0w, Rm0)
    A0 = _sc_scatter1(m0, dst2)
    h1 = _tc_node0(h0, A0, r0T, bias0_r)
    h1s = _sc_gather32(h1, src)
    m1a, m1b = _tc_msg1(h1s, ea5, W1aT, b1a_r, Wg1, Bb1w, Rm1)
    A1 = _sc_scatter2(m1a, m1b, dst2)
    h2 = _tc_node1(h1, A1, r1T, bias1_r)
    pooled = _sc_pool(h2, n2s2p, s2sp, s2gp)
    return _tc_mlp(pooled, fc1T, fb1, fc2T, fb2, fc3T, fb3)


# speedup vs baseline: 1.1815x; 1.0374x over previous
import functools

import jax
import jax.numpy as jnp
from jax import lax
from jax.experimental import pallas as pl
from jax.experimental.pallas import tpu as pltpu
from jax.experimental.pallas import tpu_sc as plsc

N = 10000
E = 160000
S2 = 2000
S1 = 400
G = 64

NPAD = 10240
EPAD = 163840
S2PAD = 2048
S1PAD = 512

NC = 2
NS = 16
NW = NC * NS

_MESH = plsc.VectorSubcoreMesh(core_axis_name="c", subcore_axis_name="s")
_SC_PARAMS = pltpu.CompilerParams(use_tc_tiling_on_sc=False)
F32 = jnp.float32


def _wid():
    return lax.axis_index("s") * NC + lax.axis_index("c")


@functools.partial(
    pl.kernel,
    out_type=jax.ShapeDtypeStruct((NPAD, 16), F32),
    mesh=_MESH,
    scratch_types=[
        pltpu.VMEM((320,), jnp.int32),
        pltpu.VMEM((320,), jnp.int32),
        pltpu.VMEM((2, 64, 16), F32),
        pltpu.SemaphoreType.DMA,
        pltpu.SemaphoreType.DMA,
        pltpu.SemaphoreType.DMA,
    ],
    compiler_params=_SC_PARAMS,
)
def _sc_embed(ztab, xtab, zidx, xidx, h0, zi_v, xi_v, r_v, zsem, asem, ssem):
    base = _wid() * 320
    pltpu.sync_copy(zidx.at[pl.ds(base, 320)], zi_v)
    pltpu.sync_copy(xidx.at[pl.ds(base, 320)], xi_v)

    def fire_zg(k):
        pltpu.async_copy(ztab.at[zi_v.at[pl.ds(k * 64, 64)]],
                         r_v.at[lax.rem(k, 2)], zsem)

    fire_zg(0)

    def step(k, _):
        b = lax.rem(k, 2)
        pltpu.make_async_copy(ztab.at[zi_v.at[pl.ds(0, 64)]],
                              r_v.at[b], zsem).wait()
        pltpu.async_copy(xtab.at[xi_v.at[pl.ds(k * 64, 64)]],
                         r_v.at[b], asem, add=True)

        @pl.when(k >= 1)
        def _():
            pltpu.make_async_copy(r_v.at[b],
                                  h0.at[pl.ds(base, 64)], ssem).wait()

        @pl.when(k < 4)
        def _():
            fire_zg(k + 1)

        pltpu.make_async_copy(xtab.at[xi_v.at[pl.ds(0, 64)]],
                              r_v.at[b], asem).wait()
        pltpu.async_copy(r_v.at[b], h0.at[pl.ds(base + k * 64, 64)], ssem)
        return 0

    lax.fori_loop(0, 5, step, 0)
    pltpu.make_async_copy(r_v.at[0], h0.at[pl.ds(base, 64)], ssem).wait()


def _make_sc_gather(C):
    @functools.partial(
        pl.kernel,
        out_type=jax.ShapeDtypeStruct((EPAD, 128), F32),
        mesh=_MESH,
        scratch_types=[
            pltpu.VMEM((5120,), jnp.int32),
            pltpu.VMEM((4, 128, C), F32),
            pltpu.SemaphoreType.DMA,
            pltpu.SemaphoreType.DMA,
        ],
        compiler_params=_SC_PARAMS,
    )
    def gather(h, src, out, i_v, r_v, gsem, ssem):
        base = _wid() * 5120
        pltpu.sync_copy(src.at[pl.ds(base, 5120)], i_v)

        def fire_g(k):
            pltpu.async_copy(h.at[i_v.at[pl.ds(k * 128, 128)]],
                             r_v.at[lax.rem(k, 4)], gsem)

        for k in range(2):
            fire_g(k)

        def step(k, _):
            b = lax.rem(k, 4)
            pltpu.make_async_copy(
                h.at[i_v.at[pl.ds(0, 128)]],
                r_v.at[b], gsem).wait()
            pltpu.async_copy(r_v.at[b],
                             out.at[pl.ds(base + k * 128, 128), pl.ds(0, C)],
                             ssem)

            @pl.when(k >= 2)
            def _():
                pltpu.make_async_copy(
                    r_v.at[0],
                    out.at[pl.ds(base, 128), pl.ds(0, C)], ssem).wait()

            @pl.when(k + 2 < 40)
            def _():
                fire_g(k + 2)

            return 0

        lax.fori_loop(0, 40, step, 0)
        for _k in range(2):
            pltpu.make_async_copy(
                r_v.at[0],
                out.at[pl.ds(base, 128), pl.ds(0, C)], ssem).wait()

    return gather


_sc_gather16 = _make_sc_gather(16)
_sc_gather32 = _make_sc_gather(32)


def _make_sc_scatter(n_in):
    rows_per_worker = NPAD // NS
    nchunk = EPAD // 128 // NS

    @functools.partial(
        pl.kernel,
        out_type=jax.ShapeDtypeStruct((NPAD, 128), F32),
        mesh=_MESH,
        scratch_types=[
            pltpu.VMEM_SHARED((NPAD, 64), F32),
            pltpu.VMEM((128, 64), F32),
            pltpu.VMEM((4, 128, 64), F32),
            pltpu.VMEM((nchunk, 128), jnp.int32),
            pltpu.SemaphoreType.DMA,
            pltpu.SemaphoreType.DMA,
        ],
        compiler_params=_SC_PARAMS,
    )
    def scatter(*args):
        msgs = args[:n_in]
        dst2 = args[n_in]
        out = args[n_in + 1]
        acc, zb, m_v, i_v, lsem, asem = args[n_in + 2:]
        cid = lax.axis_index("c")
        sid = lax.axis_index("s")

        def zrow(i, _):
            for q in range(4):
                zb[i, pl.ds(q * 16, 16)] = jnp.zeros((16,), F32)
            return 0

        lax.fori_loop(0, 128, zrow, 0)

        r0 = sid * rows_per_worker
        for j in range(rows_per_worker // 128):
            pltpu.sync_copy(zb, acc.at[pl.ds(r0 + j * 128, 128)])

        cbase = sid * nchunk
        pltpu.sync_copy(dst2.at[pl.ds(cbase, nchunk)], i_v)
        plsc.subcore_barrier()

        for msg in msgs:
            def fire_l(k):
                pltpu.async_copy(
                    msg.at[pl.ds((cbase + k) * 128, 128),
                           pl.ds(cid * 64, 64)],
                    m_v.at[lax.rem(k, 4)], lsem)

            for k in range(2):
                fire_l(k)

            def step(k, _):
                b = lax.rem(k, 4)
                pltpu.make_async_copy(
                    msg.at[pl.ds(cbase * 128, 128), pl.ds(0, 64)],
                    m_v.at[b], lsem).wait()
                pltpu.async_copy(m_v.at[b], acc.at[i_v.at[k]],
                                 asem, add=True)

                @pl.when(k >= 2)
                def _():
                    pltpu.make_async_copy(m_v.at[0],
                                          acc.at[i_v.at[0]], asem).wait()

                @pl.when(k + 2 < nchunk)
                def _():
                    fire_l(k + 2)

                return 0

            lax.fori_loop(0, nchunk, step, 0)
            for _k in range(2):
                pltpu.make_async_copy(m_v.at[0],
                                      acc.at[i_v.at[0]], asem).wait()

        plsc.subcore_barrier()
        pltpu.sync_copy(acc.at[pl.ds(r0, rows_per_worker)],
                        out.at[pl.ds(r0, rows_per_worker),
                               pl.ds(cid * 64, 64)])

    return scatter


_sc_scatter1 = _make_sc_scatter(1)
_sc_scatter2 = _make_sc_scatter(2)


@functools.partial(
    pl.kernel,
    out_type=jax.ShapeDtypeStruct((G, 128), F32),
    mesh=_MESH,
    scratch_types=[
        pltpu.VMEM_SHARED((S2PAD, 128), F32),
        pltpu.VMEM_SHARED((S2PAD, 16), F32),
        pltpu.VMEM_SHARED((S1PAD, 128), F32),
        pltpu.VMEM_SHARED((S1PAD, 16), F32),
        pltpu.VMEM_SHARED((128, 128), F32),
        pltpu.VMEM_SHARED((128, 16), F32),
        pltpu.VMEM((128, 128), F32),
        pltpu.VMEM((128, 16), F32),
        pltpu.VMEM((128, 16), F32),
        pltpu.VMEM((128,), jnp.int32),
        pltpu.VMEM((32, 128), F32),
        pltpu.VMEM((32, 16), F32),
        pltpu.VMEM((32, 16), F32),
        pltpu.VMEM((32,), jnp.int32),
        pltpu.VMEM((4, 128), F32),
        pltpu.VMEM((4, 16), F32),
    ],
    compiler_params=_SC_PARAMS,
)
def _sc_pool(h2, n2s2, s2s, s2g, out,
             sum1, cnt1, sum2, cnt2, sum3, cnt3,
             vb, cb, ob, ib, vb32, cb32, ob32, ib32, vb4, cb4):
    cid = lax.axis_index("c")
    sid = lax.axis_index("s")

    @pl.when(cid == 0)
    def _():
        def fill(i, _):
            for q in range(8):
                vb[i, pl.ds(q * 16, 16)] = jnp.zeros((16,), F32)
            cb[i] = jnp.zeros((16,), F32)
            ob[i] = jnp.ones((16,), F32)
            return 0

        lax.fori_loop(0, 128, fill, 0)

        def fill32(i, _):
            ob32[i] = jnp.ones((16,), F32)
            return 0

        lax.fori_loop(0, 32, fill32, 0)

        pltpu.sync_copy(vb, sum1.at[pl.ds(sid * 128, 128)])
        pltpu.sync_copy(cb, cnt1.at[pl.ds(sid * 128, 128)])
        pltpu.sync_copy(vb.at[pl.ds(0, 32)], sum2.at[pl.ds(sid * 32, 32)])
        pltpu.sync_copy(cb.at[pl.ds(0, 32)], cnt2.at[pl.ds(sid * 32, 32)])
        pltpu.sync_copy(vb.at[pl.ds(0, 8)], sum3.at[pl.ds(sid * 8, 8)])
        pltpu.sync_copy(cb.at[pl.ds(0, 8)], cnt3.at[pl.ds(sid * 8, 8)])
        plsc.subcore_barrier()

        def chunk(k, _):
            off = sid * 640 + k * 128
            pltpu.sync_copy(n2s2.at[pl.ds(off, 128)], ib)
            pltpu.sync_copy(h2.at[pl.ds(off, 128)], vb)
            pltpu.sync_copy(vb, sum1.at[ib], add=True)
            pltpu.sync_copy(ob, cnt1.at[ib], add=True)
            return 0

        lax.fori_loop(0, 5, chunk, 0)
        plsc.subcore_barrier()

        r0 = sid * 128
        pltpu.sync_copy(sum1.at[pl.ds(r0, 128)], vb)
        pltpu.sync_copy(cnt1.at[pl.ds(r0, 128)], cb)

        def mean1(r, _):
            inv = 1.0 / jnp.maximum(cb[r], 1.0)
            for q in range(8):
                vb[r, pl.ds(q * 16, 16)] = vb[r, pl.ds(q * 16, 16)] * inv
            return 0

        lax.fori_loop(0, 128, mean1, 0)
        pltpu.sync_copy(s2s.at[pl.ds(r0, 128)], ib)
        pltpu.sync_copy(vb, sum2.at[ib], add=True)
        pltpu.sync_copy(ob, cnt2.at[ib], add=True)
        plsc.subcore_barrier()

        r1 = sid * 32
        pltpu.sync_copy(sum2.at[pl.ds(r1, 32)], vb32)
        pltpu.sync_copy(cnt2.at[pl.ds(r1, 32)], cb32)

        def mean2(r, _):
            inv = 1.0 / jnp.maximum(cb32[r], 1.0)
            for q in range(8):
                vb32[r, pl.ds(q * 16, 16)] = vb32[r, pl.ds(q * 16, 16)] * inv
            return 0

        lax.fori_loop(0, 32, mean2, 0)
        pltpu.sync_copy(s2g.at[pl.ds(r1, 32)], ib32)
        pltpu.sync_copy(vb32, sum3.at[ib32], add=True)
        pltpu.sync_copy(ob32, cnt3.at[ib32], add=True)
        plsc.subcore_barrier()

        r2 = sid * 4
        pltpu.sync_copy(sum3.at[pl.ds(r2, 4)], vb4)
        pltpu.sync_copy(cnt3.at[pl.ds(r2, 4)], cb4)

        def mean3(r, _):
            inv = 1.0 / jnp.maximum(cb4[r], 1.0)
            for q in range(8):
                vb4[r, pl.ds(q * 16, 16)] = vb4[r, pl.ds(q * 16, 16)] * inv
            return 0

        lax.fori_loop(0, 4, mean3, 0)
        pltpu.sync_copy(vb4, out.at[pl.ds(r2, 4)])


def _make_tc_msg(cin, C, ngrp, n_out, T):
    grid = EPAD // T
    per_out = ngrp // n_out

    def body(*refs):
        hs_ref, ea_ref, WaT_ref, ba_ref, Wg_ref, Bbw_ref, Rm_ref = refs[:7]
        outs = refs[7:]
        pid = pl.program_id(0)
        hs = hs_ref[...][:, :cin]
        u = jnp.maximum(
            jnp.dot(ea_ref[...], WaT_ref[...],
                    preferred_element_type=F32) + ba_ref[...], 0.0)
        hm = jnp.dot(hs.astype(jnp.bfloat16), Rm_ref[...],
                     preferred_element_type=F32)
        u16 = u.astype(jnp.bfloat16)
        row = lax.broadcasted_iota(jnp.int32, (T, 1), 0) + pid * T
        for o in range(n_out):
            acc = None
            for gg in range(per_out):
                g = o * per_out + gg
                blk = jnp.dot(u16, Wg_ref[g], preferred_element_type=F32)
                term = blk * hm[:, g * 128:(g + 1) * 128]
                acc = term if acc is None else acc + term
            if o == 0:
                acc = acc + jnp.dot(hs, Bbw_ref[...],
                                    preferred_element_type=F32)
            outs[o][...] = jnp.where(row < E, acc, 0.0)

    return pl.pallas_call(
        body,
        grid=(grid,),
        in_specs=[
            pl.BlockSpec((T, 128), lambda i: (i, 0)),
            pl.BlockSpec((T, 5), lambda i: (i, 0)),
            pl.BlockSpec((5, 128), lambda i: (0, 0)),
            pl.BlockSpec((1, 128), lambda i: (0, 0)),
            pl.BlockSpec((ngrp, 128, 128), lambda i: (0, 0, 0)),
            pl.BlockSpec((cin, 128), lambda i: (0, 0)),
            pl.BlockSpec((cin, ngrp * 128), lambda i: (0, 0)),
        ],
        out_specs=[pl.BlockSpec((T, 128), lambda i: (i, 0))] * n_out,
        out_shape=[jax.ShapeDtypeStruct((EPAD, 128), F32)] * n_out,
    )


_tc_msg0 = _make_tc_msg(16, 32, 4, 1, 2048)
_tc_msg1 = _make_tc_msg(32, 64, 16, 2, 2048)


def _elu(v):
    return jnp.where(v > 0, v, jnp.exp(jnp.minimum(v, 0.0)) - 1.0)


def _make_tc_node(cin, C, T, out_wide):
    grid = NPAD // T
    nblk = 128 // C

    def body(h_ref, A_ref, rT_ref, b_ref, o_ref):
        pid = pl.program_id(0)
        v = jnp.dot(h_ref[...], rT_ref[...],
                    preferred_element_type=F32)
        A = A_ref[...]
        for j in range(nblk):
            v = v + A[:, j * C:(j + 1) * C]
        v = _elu(v + b_ref[...])
        row = lax.broadcasted_iota(jnp.int32, (T, 1), 0) + pid * T
        v = jnp.where(row < N, v, 0.0)
        if out_wide:
            o_ref[:, :C] = v
            o_ref[:, C:] = jnp.zeros((T, 128 - C), F32)
        else:
            o_ref[...] = v

    return pl.pallas_call(
        body,
        grid=(grid,),
        in_specs=[
            pl.BlockSpec((T, cin), lambda i: (i, 0)),
            pl.BlockSpec((T, 128), lambda i: (i, 0)),
            pl.BlockSpec((cin, C), lambda i: (0, 0)),
            pl.BlockSpec((1, C), lambda i: (0, 0)),
        ],
        out_specs=pl.BlockSpec((T, 128 if out_wide else C),
                               lambda i: (i, 0)),
        out_shape=jax.ShapeDtypeStruct((NPAD, 128 if out_wide else C), F32),
    )


_tc_node0 = _make_tc_node(16, 32, 1024, out_wide=False)
_tc_node1 = _make_tc_node(32, 64, 1024, out_wide=True)


def _tc_mlp(pooled, w1T, b1, w2T, b2, w3T, b3):
    def body(p_ref, w1_ref, b1_ref, w2_ref, b2_ref, w3_ref, b3_ref, o_ref):
        v = _elu(jnp.dot(p_ref[...][:, :64], w1_ref[...],
                         preferred_element_type=F32) + b1_ref[...])
        v = _elu(jnp.dot(v, w2_ref[...],
                         preferred_element_type=F32) + b2_ref[...])
        o_ref[...] = jnp.dot(v, w3_ref[...],
                             preferred_element_type=F32) + b3_ref[...]

    return pl.pallas_call(
        body,
        out_shape=jax.ShapeDtypeStruct((G, 1), F32),
    )(pooled, w1T, b1, w2T, b2, w3T, b3)


def kernel(x, z, edge_index, edge_attr, batch, node_to_subgraph2,
           subgraph2_to_subgraph, subgraph_to_graph,
           z_table, nt_table,
           W0a, b0a, W0b, b0b, root0, bias0,
           W1a, b1a, W1b, b1b, root1, bias1,
           fc1_w, fc1_b, fc2_w, fc2_b, fc3_w, fc3_b):
    i32 = jnp.int32

    ztab = jnp.pad(z_table, ((0, 0), (0, 8)))
    xtab = jnp.pad(nt_table, ((0, 3), (0, 8)))
    xtab = xtab.at[:5, 8].set(jnp.arange(5, dtype=F32))

    zidx = jnp.pad(z.astype(i32), (0, NPAD - N))
    xidx = jnp.pad(x.astype(i32), (0, NPAD - N))
    src = jnp.pad(edge_index[0].astype(i32), (0, EPAD - E))
    dst = jnp.pad(edge_index[1].astype(i32), (0, EPAD - E))
    dst2 = dst.reshape(EPAD // 128, 128)
    ea5 = jnp.pad(edge_attr, ((0, EPAD - E), (0, 0)))

    W0aT = W0a.T
    W1aT = W1a.T
    b0a_r = b0a.reshape(1, 128)
    b1a_r = b1a.reshape(1, 128)
    Wg0 = jnp.pad(W0b.reshape(9, 32, 128), ((0, 7), (0, 0), (0, 0)))
    Wg0 = Wg0.transpose(0, 2, 1).reshape(4, 4, 128, 32)
    Wg0 = Wg0.transpose(0, 2, 1, 3).reshape(4, 128, 128)
    Wg1 = W1b.reshape(32, 64, 128).transpose(0, 2, 1).reshape(16, 2, 128, 64)
    Wg1 = Wg1.transpose(0, 2, 1, 3).reshape(16, 128, 128)
    Wg0 = Wg0.astype(jnp.bfloat16)
    Wg1 = Wg1.astype(jnp.bfloat16)
    Bb0w = jnp.pad(jnp.pad(b0b.reshape(9, 32), ((0, 7), (0, 0))),
                   ((0, 0), (0, 96)))
    Bb1w = jnp.pad(b1b.reshape(32, 64), ((0, 0), (0, 64)))
    Rm0 = jnp.kron(jnp.eye(16, dtype=F32), jnp.ones((1, 32), F32))
    Rm1 = jnp.kron(jnp.eye(32, dtype=F32), jnp.ones((1, 64), F32))
    Rm0 = Rm0.astype(jnp.bfloat16)
    Rm1 = Rm1.astype(jnp.bfloat16)

    r0T = jnp.pad(root0.T, ((0, 7), (0, 0)))
    r1T = root1.T
    bias0_r = bias0.reshape(1, 32)
    bias1_r = bias1.reshape(1, 64)

    n2s2p = jnp.pad(node_to_subgraph2.astype(i32), (0, NPAD - N),
                    constant_values=S2)
    s2sp = jnp.pad(subgraph2_to_subgraph.astype(i32), (0, S2PAD - S2),
                   constant_values=S1)
    s2gp = jnp.pad(subgraph_to_graph.astype(i32), (0, S1PAD - S1),
                   constant_values=G)

    fc1T = fc1_w.T
    fc2T = fc2_w.T
    fc3T = fc3_w.T
    fb1 = fc1_b.reshape(1, 32)
    fb2 = fc2_b.reshape(1, 16)
    fb3 = fc3_b.reshape(1, 1)

    h0 = _sc_embed(ztab, xtab, zidx, xidx)
    h0s = _sc_gather16(h0, src)
    (m0,) = _tc_msg0(h0s, ea5, W0aT, b0a_r, Wg0, Bb0w, Rm0)
    A0 = _sc_scatter1(m0, dst2)
    h1 = _tc_node0(h0, A0, r0T, bias0_r)
    h1s = _sc_gather32(h1, src)
    m1a, m1b = _tc_msg1(h1s, ea5, W1aT, b1a_r, Wg1, Bb1w, Rm1)
    A1 = _sc_scatter2(m1a, m1b, dst2)
    h2 = _tc_node1(h1, A1, r1T, bias1_r)
    pooled = _sc_pool(h2, n2s2p, s2sp, s2gp)
    return _tc_mlp(pooled, fc1T, fb1, fc2T, fb2, fc3T, fb3)


# speedup vs baseline: 1.2194x; 1.0320x over previous
import functools

import jax
import jax.numpy as jnp
from jax import lax
from jax.experimental import pallas as pl
from jax.experimental.pallas import tpu as pltpu
from jax.experimental.pallas import tpu_sc as plsc

N = 10000
E = 160000
S2 = 2000
S1 = 400
G = 64

NPAD = 10240
EPAD = 163840
S2PAD = 2048
S1PAD = 512

NC = 2
NS = 16
NW = NC * NS

_MESH = plsc.VectorSubcoreMesh(core_axis_name="c", subcore_axis_name="s")
_SC_PARAMS = pltpu.CompilerParams(use_tc_tiling_on_sc=False)
F32 = jnp.float32


def _wid():
    return lax.axis_index("s") * NC + lax.axis_index("c")


@functools.partial(
    pl.kernel,
    out_type=jax.ShapeDtypeStruct((NPAD, 16), F32),
    mesh=_MESH,
    scratch_types=[
        pltpu.VMEM((320,), jnp.int32),
        pltpu.VMEM((320,), jnp.int32),
        pltpu.VMEM((2, 64, 16), F32),
        pltpu.SemaphoreType.DMA,
        pltpu.SemaphoreType.DMA,
        pltpu.SemaphoreType.DMA,
    ],
    compiler_params=_SC_PARAMS,
)
def _sc_embed(ztab, xtab, zidx, xidx, h0, zi_v, xi_v, r_v, zsem, asem, ssem):
    base = _wid() * 320
    pltpu.sync_copy(zidx.at[pl.ds(base, 320)], zi_v)
    pltpu.sync_copy(xidx.at[pl.ds(base, 320)], xi_v)

    def fire_zg(k):
        pltpu.async_copy(ztab.at[zi_v.at[pl.ds(k * 64, 64)]],
                         r_v.at[lax.rem(k, 2)], zsem)

    fire_zg(0)

    def step(k, _):
        b = lax.rem(k, 2)
        pltpu.make_async_copy(ztab.at[zi_v.at[pl.ds(0, 64)]],
                              r_v.at[b], zsem).wait()
        pltpu.async_copy(xtab.at[xi_v.at[pl.ds(k * 64, 64)]],
                         r_v.at[b], asem, add=True)

        @pl.when(k >= 1)
        def _():
            pltpu.make_async_copy(r_v.at[b],
                                  h0.at[pl.ds(base, 64)], ssem).wait()

        @pl.when(k < 4)
        def _():
            fire_zg(k + 1)

        pltpu.make_async_copy(xtab.at[xi_v.at[pl.ds(0, 64)]],
                              r_v.at[b], asem).wait()
        pltpu.async_copy(r_v.at[b], h0.at[pl.ds(base + k * 64, 64)], ssem)
        return 0

    lax.fori_loop(0, 5, step, 0)
    pltpu.make_async_copy(r_v.at[0], h0.at[pl.ds(base, 64)], ssem).wait()


def _make_sc_gather(C):
    @functools.partial(
        pl.kernel,
        out_type=jax.ShapeDtypeStruct((EPAD, 128), F32),
        mesh=_MESH,
        scratch_types=[
            pltpu.VMEM((5120,), jnp.int32),
            pltpu.VMEM((4, 128, C), F32),
            pltpu.SemaphoreType.DMA,
            pltpu.SemaphoreType.DMA,
        ],
        compiler_params=_SC_PARAMS,
    )
    def gather(h, src, out, i_v, r_v, gsem, ssem):
        base = _wid() * 5120
        pltpu.sync_copy(src.at[pl.ds(base, 5120)], i_v)

        def fire_g(k):
            pltpu.async_copy(h.at[i_v.at[pl.ds(k * 128, 128)]],
                             r_v.at[lax.rem(k, 4)], gsem)

        for k in range(2):
            fire_g(k)

        def step(k, _):
            b = lax.rem(k, 4)
            pltpu.make_async_copy(
                h.at[i_v.at[pl.ds(0, 128)]],
                r_v.at[b], gsem).wait()
            pltpu.async_copy(r_v.at[b],
                             out.at[pl.ds(base + k * 128, 128), pl.ds(0, C)],
                             ssem)

            @pl.when(k >= 2)
            def _():
                pltpu.make_async_copy(
                    r_v.at[0],
                    out.at[pl.ds(base, 128), pl.ds(0, C)], ssem).wait()

            @pl.when(k + 2 < 40)
            def _():
                fire_g(k + 2)

            return 0

        lax.fori_loop(0, 40, step, 0)
        for _k in range(2):
            pltpu.make_async_copy(
                r_v.at[0],
                out.at[pl.ds(base, 128), pl.ds(0, C)], ssem).wait()

    return gather


_sc_gather16 = _make_sc_gather(16)
_sc_gather32 = _make_sc_gather(32)


def _make_sc_scatter(n_in):
    rows_per_worker = NPAD // NS
    nchunk = EPAD // 128 // NS

    @functools.partial(
        pl.kernel,
        out_type=jax.ShapeDtypeStruct((NPAD, 128), F32),
        mesh=_MESH,
        scratch_types=[
            pltpu.VMEM_SHARED((NPAD, 64), F32),
            pltpu.VMEM((128, 64), F32),
            pltpu.VMEM((4, 128, 64), F32),
            pltpu.VMEM((nchunk, 128), jnp.int32),
            pltpu.SemaphoreType.DMA,
            pltpu.SemaphoreType.DMA,
        ],
        compiler_params=_SC_PARAMS,
    )
    def scatter(*args):
        msgs = args[:n_in]
        dst2 = args[n_in]
        out = args[n_in + 1]
        acc, zb, m_v, i_v, lsem, asem = args[n_in + 2:]
        cid = lax.axis_index("c")
        sid = lax.axis_index("s")

        def zrow(i, _):
            for q in range(4):
                zb[i, pl.ds(q * 16, 16)] = jnp.zeros((16,), F32)
            return 0

        lax.fori_loop(0, 128, zrow, 0)

        r0 = sid * rows_per_worker
        for j in range(rows_per_worker // 128):
            pltpu.sync_copy(zb, acc.at[pl.ds(r0 + j * 128, 128)])

        cbase = sid * nchunk
        pltpu.sync_copy(dst2.at[pl.ds(cbase, nchunk)], i_v)
        plsc.subcore_barrier()

        for msg in msgs:
            def fire_l(k):
                pltpu.async_copy(
                    msg.at[pl.ds((cbase + k) * 128, 128),
                           pl.ds(cid * 64, 64)],
                    m_v.at[lax.rem(k, 4)], lsem)

            for k in range(2):
                fire_l(k)

            def step(k, _):
                b = lax.rem(k, 4)
                pltpu.make_async_copy(
                    msg.at[pl.ds(cbase * 128, 128), pl.ds(0, 64)],
                    m_v.at[b], lsem).wait()
                pltpu.async_copy(m_v.at[b], acc.at[i_v.at[k]],
                                 asem, add=True)

                @pl.when(k >= 2)
                def _():
                    pltpu.make_async_copy(m_v.at[0],
                                          acc.at[i_v.at[0]], asem).wait()

                @pl.when(k + 2 < nchunk)
                def _():
                    fire_l(k + 2)

                return 0

            lax.fori_loop(0, nchunk, step, 0)
            for _k in range(2):
                pltpu.make_async_copy(m_v.at[0],
                                      acc.at[i_v.at[0]], asem).wait()

        plsc.subcore_barrier()
        pltpu.sync_copy(acc.at[pl.ds(r0, rows_per_worker)],
                        out.at[pl.ds(r0, rows_per_worker),
                               pl.ds(cid * 64, 64)])

    return scatter


_sc_scatter1 = _make_sc_scatter(1)
_sc_scatter2 = _make_sc_scatter(2)


@functools.partial(
    pl.kernel,
    out_type=jax.ShapeDtypeStruct((G, 128), F32),
    mesh=_MESH,
    scratch_types=[
        pltpu.VMEM_SHARED((S2PAD, 128), F32),
        pltpu.VMEM_SHARED((S2PAD, 16), F32),
        pltpu.VMEM_SHARED((S1PAD, 128), F32),
        pltpu.VMEM_SHARED((S1PAD, 16), F32),
        pltpu.VMEM_SHARED((128, 128), F32),
        pltpu.VMEM_SHARED((128, 16), F32),
        pltpu.VMEM((128, 128), F32),
        pltpu.VMEM((128, 16), F32),
        pltpu.VMEM((128, 16), F32),
        pltpu.VMEM((128,), jnp.int32),
        pltpu.VMEM((32, 128), F32),
        pltpu.VMEM((32, 16), F32),
        pltpu.VMEM((32, 16), F32),
        pltpu.VMEM((32,), jnp.int32),
        pltpu.VMEM((4, 128), F32),
        pltpu.VMEM((4, 16), F32),
    ],
    compiler_params=_SC_PARAMS,
)
def _sc_pool(h2, n2s2, s2s, s2g, out,
             sum1, cnt1, sum2, cnt2, sum3, cnt3,
             vb, cb, ob, ib, vb32, cb32, ob32, ib32, vb4, cb4):
    cid = lax.axis_index("c")
    sid = lax.axis_index("s")

    @pl.when(cid == 0)
    def _():
        def fill(i, _):
            for q in range(8):
                vb[i, pl.ds(q * 16, 16)] = jnp.zeros((16,), F32)
            cb[i] = jnp.zeros((16,), F32)
            ob[i] = jnp.ones((16,), F32)
            return 0

        lax.fori_loop(0, 128, fill, 0)

        def fill32(i, _):
            ob32[i] = jnp.ones((16,), F32)
            return 0

        lax.fori_loop(0, 32, fill32, 0)

        pltpu.sync_copy(vb, sum1.at[pl.ds(sid * 128, 128)])
        pltpu.sync_copy(cb, cnt1.at[pl.ds(sid * 128, 128)])
        pltpu.sync_copy(vb.at[pl.ds(0, 32)], sum2.at[pl.ds(sid * 32, 32)])
        pltpu.sync_copy(cb.at[pl.ds(0, 32)], cnt2.at[pl.ds(sid * 32, 32)])
        pltpu.sync_copy(vb.at[pl.ds(0, 8)], sum3.at[pl.ds(sid * 8, 8)])
        pltpu.sync_copy(cb.at[pl.ds(0, 8)], cnt3.at[pl.ds(sid * 8, 8)])
        plsc.subcore_barrier()

        def chunk(k, _):
            off = sid * 640 + k * 128
            pltpu.sync_copy(n2s2.at[pl.ds(off, 128)], ib)
            pltpu.sync_copy(h2.at[pl.ds(off, 128)], vb)
            pltpu.sync_copy(vb, sum1.at[ib], add=True)
            pltpu.sync_copy(ob, cnt1.at[ib], add=True)
            return 0

        lax.fori_loop(0, 5, chunk, 0)
        plsc.subcore_barrier()

        r0 = sid * 128
        pltpu.sync_copy(sum1.at[pl.ds(r0, 128)], vb)
        pltpu.sync_copy(cnt1.at[pl.ds(r0, 128)], cb)

        def mean1(r, _):
            inv = 1.0 / jnp.maximum(cb[r], 1.0)
            for q in range(8):
                vb[r, pl.ds(q * 16, 16)] = vb[r, pl.ds(q * 16, 16)] * inv
            return 0

        lax.fori_loop(0, 128, mean1, 0)
        pltpu.sync_copy(s2s.at[pl.ds(r0, 128)], ib)
        pltpu.sync_copy(vb, sum2.at[ib], add=True)
        pltpu.sync_copy(ob, cnt2.at[ib], add=True)
        plsc.subcore_barrier()

        r1 = sid * 32
        pltpu.sync_copy(sum2.at[pl.ds(r1, 32)], vb32)
        pltpu.sync_copy(cnt2.at[pl.ds(r1, 32)], cb32)

        def mean2(r, _):
            inv = 1.0 / jnp.maximum(cb32[r], 1.0)
            for q in range(8):
                vb32[r, pl.ds(q * 16, 16)] = vb32[r, pl.ds(q * 16, 16)] * inv
            return 0

        lax.fori_loop(0, 32, mean2, 0)
        pltpu.sync_copy(s2g.at[pl.ds(r1, 32)], ib32)
        pltpu.sync_copy(vb32, sum3.at[ib32], add=True)
        pltpu.sync_copy(ob32, cnt3.at[ib32], add=True)
        plsc.subcore_barrier()

        r2 = sid * 4
        pltpu.sync_copy(sum3.at[pl.ds(r2, 4)], vb4)
        pltpu.sync_copy(cnt3.at[pl.ds(r2, 4)], cb4)

        def mean3(r, _):
            inv = 1.0 / jnp.maximum(cb4[r], 1.0)
            for q in range(8):
                vb4[r, pl.ds(q * 16, 16)] = vb4[r, pl.ds(q * 16, 16)] * inv
            return 0

        lax.fori_loop(0, 4, mean3, 0)
        pltpu.sync_copy(vb4, out.at[pl.ds(r2, 4)])


def _make_tc_msg(cin, C, ngrp, n_out, T):
    grid = EPAD // T
    per_out = ngrp // n_out

    def body(*refs):
        hs_ref, ea_ref, WaT_ref, ba_ref, Wg_ref, Bbw_ref, Rm_ref = refs[:7]
        outs = refs[7:]
        pid = pl.program_id(0)
        hs = hs_ref[...][:, :cin]
        u = jnp.maximum(
            jnp.dot(ea_ref[...], WaT_ref[...],
                    preferred_element_type=F32) + ba_ref[...], 0.0)
        hm = jnp.dot(hs.astype(jnp.bfloat16), Rm_ref[...],
                     preferred_element_type=F32)
        u16 = u.astype(jnp.bfloat16)
        row = lax.broadcasted_iota(jnp.int32, (T, 1), 0) + pid * T
        for o in range(n_out):
            acc = None
            for gg in range(per_out):
                g = o * per_out + gg
                blk = jnp.dot(u16, Wg_ref[g], preferred_element_type=F32)
                term = blk * hm[:, g * 128:(g + 1) * 128]
                acc = term if acc is None else acc + term
            if o == 0:
                acc = acc + jnp.dot(hs, Bbw_ref[...],
                                    preferred_element_type=F32)
            outs[o][...] = jnp.where(row < E, acc, 0.0)

    return pl.pallas_call(
        body,
        grid=(grid,),
        in_specs=[
            pl.BlockSpec((T, 128), lambda i: (i, 0)),
            pl.BlockSpec((T, 5), lambda i: (i, 0)),
            pl.BlockSpec((5, 128), lambda i: (0, 0)),
            pl.BlockSpec((1, 128), lambda i: (0, 0)),
            pl.BlockSpec((ngrp, 128, 128), lambda i: (0, 0, 0)),
            pl.BlockSpec((cin, 128), lambda i: (0, 0)),
            pl.BlockSpec((cin, ngrp * 128), lambda i: (0, 0)),
        ],
        out_specs=[pl.BlockSpec((T, 128), lambda i: (i, 0))] * n_out,
        out_shape=[jax.ShapeDtypeStruct((EPAD, 128), F32)] * n_out,
    )


_tc_msg0 = _make_tc_msg(16, 32, 4, 1, 4096)
_tc_msg1 = _make_tc_msg(32, 64, 16, 2, 4096)


def _elu(v):
    return jnp.where(v > 0, v, jnp.exp(jnp.minimum(v, 0.0)) - 1.0)


def _make_tc_node(cin, C, T, out_wide):
    grid = NPAD // T
    nblk = 128 // C

    def body(h_ref, A_ref, rT_ref, b_ref, o_ref):
        pid = pl.program_id(0)
        v = jnp.dot(h_ref[...], rT_ref[...],
                    preferred_element_type=F32)
        A = A_ref[...]
        for j in range(nblk):
            v = v + A[:, j * C:(j + 1) * C]
        v = _elu(v + b_ref[...])
        row = lax.broadcasted_iota(jnp.int32, (T, 1), 0) + pid * T
        v = jnp.where(row < N, v, 0.0)
        if out_wide:
            o_ref[:, :C] = v
            o_ref[:, C:] = jnp.zeros((T, 128 - C), F32)
        else:
            o_ref[...] = v

    return pl.pallas_call(
        body,
        grid=(grid,),
        in_specs=[
            pl.BlockSpec((T, cin), lambda i: (i, 0)),
            pl.BlockSpec((T, 128), lambda i: (i, 0)),
            pl.BlockSpec((cin, C), lambda i: (0, 0)),
            pl.BlockSpec((1, C), lambda i: (0, 0)),
        ],
        out_specs=pl.BlockSpec((T, 128 if out_wide else C),
                               lambda i: (i, 0)),
        out_shape=jax.ShapeDtypeStruct((NPAD, 128 if out_wide else C), F32),
    )


_tc_node0 = _make_tc_node(16, 32, 1024, out_wide=False)
_tc_node1 = _make_tc_node(32, 64, 1024, out_wide=True)


def _tc_mlp(pooled, w1T, b1, w2T, b2, w3T, b3):
    def body(p_ref, w1_ref, b1_ref, w2_ref, b2_ref, w3_ref, b3_ref, o_ref):
        v = _elu(jnp.dot(p_ref[...][:, :64], w1_ref[...],
                         preferred_element_type=F32) + b1_ref[...])
        v = _elu(jnp.dot(v, w2_ref[...],
                         preferred_element_type=F32) + b2_ref[...])
        o_ref[...] = jnp.dot(v, w3_ref[...],
                             preferred_element_type=F32) + b3_ref[...]

    return pl.pallas_call(
        body,
        out_shape=jax.ShapeDtypeStruct((G, 1), F32),
    )(pooled, w1T, b1, w2T, b2, w3T, b3)


def kernel(x, z, edge_index, edge_attr, batch, node_to_subgraph2,
           subgraph2_to_subgraph, subgraph_to_graph,
           z_table, nt_table,
           W0a, b0a, W0b, b0b, root0, bias0,
           W1a, b1a, W1b, b1b, root1, bias1,
           fc1_w, fc1_b, fc2_w, fc2_b, fc3_w, fc3_b):
    i32 = jnp.int32

    ztab = jnp.pad(z_table, ((0, 0), (0, 8)))
    xtab = jnp.pad(nt_table, ((0, 3), (0, 8)))
    xtab = xtab.at[:5, 8].set(jnp.arange(5, dtype=F32))

    zidx = jnp.pad(z.astype(i32), (0, NPAD - N))
    xidx = jnp.pad(x.astype(i32), (0, NPAD - N))
    src = jnp.pad(edge_index[0].astype(i32), (0, EPAD - E))
    dst = jnp.pad(edge_index[1].astype(i32), (0, EPAD - E))
    dst2 = dst.reshape(EPAD // 128, 128)
    ea5 = jnp.pad(edge_attr, ((0, EPAD - E), (0, 0)))

    W0aT = W0a.T
    W1aT = W1a.T
    b0a_r = b0a.reshape(1, 128)
    b1a_r = b1a.reshape(1, 128)
    Wg0 = jnp.pad(W0b.reshape(9, 32, 128), ((0, 7), (0, 0), (0, 0)))
    Wg0 = Wg0.transpose(0, 2, 1).reshape(4, 4, 128, 32)
    Wg0 = Wg0.transpose(0, 2, 1, 3).reshape(4, 128, 128)
    Wg1 = W1b.reshape(32, 64, 128).transpose(0, 2, 1).reshape(16, 2, 128, 64)
    Wg1 = Wg1.transpose(0, 2, 1, 3).reshape(16, 128, 128)
    Wg0 = Wg0.astype(jnp.bfloat16)
    Wg1 = Wg1.astype(jnp.bfloat16)
    Bb0w = jnp.pad(jnp.pad(b0b.reshape(9, 32), ((0, 7), (0, 0))),
                   ((0, 0), (0, 96)))
    Bb1w = jnp.pad(b1b.reshape(32, 64), ((0, 0), (0, 64)))
    Rm0 = jnp.kron(jnp.eye(16, dtype=F32), jnp.ones((1, 32), F32))
    Rm1 = jnp.kron(jnp.eye(32, dtype=F32), jnp.ones((1, 64), F32))
    Rm0 = Rm0.astype(jnp.bfloat16)
    Rm1 = Rm1.astype(jnp.bfloat16)

    r0T = jnp.pad(root0.T, ((0, 7), (0, 0)))
    r1T = root1.T
    bias0_r = bias0.reshape(1, 32)
    bias1_r = bias1.reshape(1, 64)

    n2s2p = jnp.pad(node_to_subgraph2.astype(i32), (0, NPAD - N),
                    constant_values=S2)
    s2sp = jnp.pad(subgraph2_to_subgraph.astype(i32), (0, S2PAD - S2),
                   constant_values=S1)
    s2gp = jnp.pad(subgraph_to_graph.astype(i32), (0, S1PAD - S1),
                   constant_values=G)

    fc1T = fc1_w.T
    fc2T = fc2_w.T
    fc3T = fc3_w.T
    fb1 = fc1_b.reshape(1, 32)
    fb2 = fc2_b.reshape(1, 16)
    fb3 = fc3_b.reshape(1, 1)

    h0 = _sc_embed(ztab, xtab, zidx, xidx)
    h0s = _sc_gather16(h0, src)
    (m0,) = _tc_msg0(h0s, ea5, W0aT, b0a_r, Wg0, Bb0w, Rm0)
    A0 = _sc_scatter1(m0, dst2)
    h1 = _tc_node0(h0, A0, r0T, bias0_r)
    h1s = _sc_gather32(h1, src)
    m1a, m1b = _tc_msg1(h1s, ea5, W1aT, b1a_r, Wg1, Bb1w, Rm1)
    A1 = _sc_scatter2(m1a, m1b, dst2)
    h2 = _tc_node1(h1, A1, r1T, bias1_r)
    pooled = _sc_pool(h2, n2s2p, s2sp, s2gp)
    return _tc_mlp(pooled, fc1T, fb1, fc2T, fb2, fc3T, fb3)
